# Initial kernel scaffold; baseline (speedup 1.0000x reference)
import jax
import jax.numpy as jnp
from jax.experimental import pallas as pl


def kernel(x, edge_index, Wl1, bl1, Wr1, Wl2, bl2, Wr2, W1, b1, W2, b2):
    raise NotImplementedError("write your pallas kernel here")


# speedup vs baseline: 6.8160x; 6.8160x over previous
import functools

import jax
import jax.numpy as jnp
from jax import lax
from jax.experimental import pallas as pl
from jax.experimental.pallas import tpu as pltpu
from jax.experimental.pallas import tpu_sc as plsc

N = 10000
E = 320000
D = 128
EPS = 1e-5

NC = 2
NS = 16
NW = NC * NS
C = 128
NCHUNK = E // C
CHUNK_ITERS = -(-NCHUNK // NW)
N_PAD = 10240
ROWS_PER_TILE = N_PAD // NS


@functools.cache
def _make_sc_agg(with_cnt):
    mesh = plsc.VectorSubcoreMesh(
        core_axis_name="c", subcore_axis_name="s", num_cores=NC, num_subcores=NS
    )
    out_type = [jax.ShapeDtypeStruct((NC * N_PAD, D), jnp.float32)]
    scratch = [
        pltpu.VMEM_SHARED((N_PAD, D), jnp.float32),
        pltpu.VMEM((C,), jnp.int32),
        pltpu.VMEM((C,), jnp.int32),
        pltpu.VMEM((C, D), jnp.float32),
        pltpu.SemaphoreType.DMA,
    ]
    if with_cnt:
        out_type.append(jax.ShapeDtypeStruct((NC * N_PAD,), jnp.float32))
        scratch += [
            pltpu.VMEM_SHARED((N_PAD,), jnp.float32),
            pltpu.VMEM((C,), jnp.float32),
            pltpu.VMEM((C,), jnp.float32),
        ]

    def body(x_hbm, src_hbm, dst_hbm, zrow_hbm, zcnt_hbm, ones_hbm, *rest):
        if with_cnt:
            agg_out, cnt_out, acc_sh, srcv, dstv, rows, sem, cnt_sh, onesv, cntb = rest
        else:
            agg_out, acc_sh, srcv, dstv, rows, sem = rest
        c = lax.axis_index("c")
        s = lax.axis_index("s")
        wid = c * NS + s
        base_row = s * ROWS_PER_TILE

        pltpu.sync_copy(zrow_hbm, rows)
        if with_cnt:
            pltpu.sync_copy(zcnt_hbm, cntb)
            pltpu.sync_copy(ones_hbm, onesv)
        for j in range(ROWS_PER_TILE // C):
            pltpu.sync_copy(rows, acc_sh.at[pl.ds(base_row + j * C, C)])
            if with_cnt:
                pltpu.sync_copy(cntb, cnt_sh.at[pl.ds(base_row + j * C, C)])
        plsc.subcore_barrier()

        start = (wid * NCHUNK) // NW
        end = ((wid + 1) * NCHUNK) // NW

        def step(cid, carry):
            off = cid * C
            pltpu.sync_copy(src_hbm.at[pl.ds(off, C)], srcv)
            pltpu.sync_copy(dst_hbm.at[pl.ds(off, C)], dstv)
            pltpu.async_copy(x_hbm.at[srcv], rows, sem).wait()
            pltpu.sync_copy(rows, acc_sh.at[dstv], add=True)
            if with_cnt:
                pltpu.sync_copy(onesv, cnt_sh.at[dstv], add=True)
            return carry

        lax.fori_loop(start, end, step, 0)
        plsc.subcore_barrier()

        out_base = c * N_PAD + base_row
        for j in range(ROWS_PER_TILE // C):
            pltpu.sync_copy(acc_sh.at[pl.ds(base_row + j * C, C)], rows)
            pltpu.sync_copy(rows, agg_out.at[pl.ds(out_base + j * C, C)])
            if with_cnt:
                pltpu.sync_copy(cnt_sh.at[pl.ds(base_row + j * C, C)], cntb)
                pltpu.sync_copy(cntb, cnt_out.at[pl.ds(out_base + j * C, C)])

    return pl.kernel(body, out_type=out_type, mesh=mesh, scratch_types=scratch)


def _bn_relu(h):
    m = jnp.mean(h, axis=0, keepdims=True)
    d = h - m
    v = jnp.mean(d * d, axis=0, keepdims=True)
    return jnp.maximum(d * lax.rsqrt(v + EPS), 0.0)


def _tc_layer1(aggp_ref, cntp_ref, x_ref, wlT_ref, bl_ref, wrT_ref, o_ref):
    a = aggp_ref[pl.ds(0, N), :] + aggp_ref[pl.ds(N_PAD, N), :]
    cnt1 = cntp_ref[pl.ds(0, N)] + cntp_ref[pl.ds(N_PAD, N)]
    cnt = jnp.maximum(cnt1.reshape(N, 1), 1.0)
    agg = a / cnt
    h = (
        jnp.dot(agg, wlT_ref[...], preferred_element_type=jnp.float32)
        + bl_ref[...]
        + jnp.dot(x_ref[...], wrT_ref[...], preferred_element_type=jnp.float32)
    )
    o_ref[...] = _bn_relu(h)


def _tc_layer2_head(
    aggp_ref, cntp_ref, h1_ref, wlT_ref, bl_ref, wrT_ref,
    w1T_ref, b1_ref, w2T_ref, b2_ref, o_ref
):
    a = aggp_ref[pl.ds(0, N), :] + aggp_ref[pl.ds(N_PAD, N), :]
    cnt1 = cntp_ref[pl.ds(0, N)] + cntp_ref[pl.ds(N_PAD, N)]
    cnt = jnp.maximum(cnt1.reshape(N, 1), 1.0)
    agg = a / cnt
    h = (
        jnp.dot(agg, wlT_ref[...], preferred_element_type=jnp.float32)
        + bl_ref[...]
        + jnp.dot(h1_ref[...], wrT_ref[...], preferred_element_type=jnp.float32)
    )
    h = _bn_relu(h)
    h = jnp.dot(h, w1T_ref[...], preferred_element_type=jnp.float32) + b1_ref[...]
    h = _bn_relu(h)
    o_ref[...] = (
        jnp.dot(h, w2T_ref[...], preferred_element_type=jnp.float32) + b2_ref[...]
    )


def kernel(x, edge_index, Wl1, bl1, Wr1, Wl2, bl2, Wr2, W1, b1, W2, b2):
    _sc_agg_cnt = _make_sc_agg(with_cnt=True)
    _sc_agg = _make_sc_agg(with_cnt=False)
    src = edge_index[0]
    dst = edge_index[1]
    zrow = jnp.zeros((C, D), jnp.float32)
    zcnt = jnp.zeros((C,), jnp.float32)
    ones = jnp.ones((C,), jnp.float32)

    aggp1, cntp = _sc_agg_cnt(x, src, dst, zrow, zcnt, ones)

    h1 = pl.pallas_call(
        _tc_layer1,
        out_shape=jax.ShapeDtypeStruct((N, D), jnp.float32),
    )(aggp1, cntp, x, Wl1.T, bl1.reshape(1, D), Wr1.T)

    (aggp2,) = _sc_agg(h1, src, dst, zrow, zcnt, ones)

    w2T = jnp.zeros((D, 8), jnp.float32).at[:, :2].set(W2.T)
    b2p = jnp.zeros((1, 8), jnp.float32).at[:, :2].set(b2.reshape(1, 2))
    out8 = pl.pallas_call(
        _tc_layer2_head,
        out_shape=jax.ShapeDtypeStruct((N, 8), jnp.float32),
    )(
        aggp2, cntp, h1, Wl2.T, bl2.reshape(1, D), Wr2.T,
        W1.T, b1.reshape(1, D), w2T, b2p,
    )
    return out8[:, :2]
